# speedup vs baseline: 7.5345x; 7.5345x over previous
import functools
import math

import jax
import jax.numpy as jnp
from jax import lax
from jax.experimental import pallas as pl
from jax.experimental.pallas import tpu as pltpu
from jax.experimental.pallas import tpu_sc as plsc

_NUM_WORKERS = 32
_LANES = 16


@functools.partial(jax.jit, static_argnums=(5, 6, 7))
def _sc_lookup(tokens_flat, pidx, token_table, persp_table, pe_slice, batch, seq, d):
    n = batch * seq
    rows_per_worker = batch // _NUM_WORKERS
    scale = math.sqrt(float(d))
    chunks = []
    off = 0
    while off < seq:
        sz = min(128, seq - off)
        chunks.append((off, sz))
        off += sz

    mesh = plsc.VectorSubcoreMesh(core_axis_name="c", subcore_axis_name="s")

    def body(tok_hbm, pidx_hbm, tab_hbm, persp_hbm, pe_hbm, out_hbm,
             idx_v, pidx_v, persp_v, addvec_v, rows_v, psem, gsem):
        cid = lax.axis_index("c")
        sid = lax.axis_index("s")
        wid = sid * 2 + cid

        pltpu.sync_copy(pe_hbm, addvec_v)
        pltpu.sync_copy(pidx_hbm, pidx_v)
        pltpu.async_copy(persp_hbm.at[pidx_v], persp_v, psem).wait()
        pregs = [persp_v[0, pl.ds(j * _LANES, _LANES)] for j in range(d // _LANES)]

        @pl.loop(0, seq)
        def _fold(s):
            for j in range(d // _LANES):
                sl = pl.ds(j * _LANES, _LANES)
                addvec_v[s, sl] = addvec_v[s, sl] + pregs[j]

        @pl.loop(0, rows_per_worker)
        def _row(b):
            r0 = (wid * rows_per_worker + b) * seq
            pltpu.sync_copy(tok_hbm.at[pl.ds(r0, seq)], idx_v)
            copies = [
                pltpu.async_copy(
                    tab_hbm.at[idx_v.at[pl.ds(coff, csz)]],
                    rows_v.at[pl.ds(coff, csz)],
                    gsem,
                )
                for coff, csz in chunks
            ]
            for c in copies:
                c.wait()

            @pl.loop(0, seq)
            def _fma(s):
                for j in range(d // _LANES):
                    sl = pl.ds(j * _LANES, _LANES)
                    rows_v[s, sl] = rows_v[s, sl] * scale + addvec_v[s, sl]

            pltpu.sync_copy(rows_v, out_hbm.at[pl.ds(r0, seq)])

    fn = pl.kernel(
        body,
        out_type=jax.ShapeDtypeStruct((n, d), jnp.float32),
        mesh=mesh,
        scratch_types=[
            pltpu.VMEM((seq,), jnp.int32),
            pltpu.VMEM((8,), jnp.int32),
            pltpu.VMEM((8, d), jnp.float32),
            pltpu.VMEM((seq, d), jnp.float32),
            pltpu.VMEM((seq, d), jnp.float32),
            pltpu.SemaphoreType.DMA,
            pltpu.SemaphoreType.DMA,
        ],
    )
    return fn(tokens_flat, pidx, token_table, persp_table, pe_slice)


def kernel(tokens, perspective, token_table, persp_table, pe):
    batch, seq = tokens.shape
    d = token_table.shape[1]
    tokens_flat = tokens.reshape(batch * seq)
    pe_slice = pe[0, :seq, :]
    pidx = jnp.full((8,), perspective, dtype=jnp.int32)
    out = _sc_lookup(tokens_flat, pidx, token_table, persp_table, pe_slice,
                     batch, seq, d)
    return out.reshape(batch, seq, d)
